# Initial kernel scaffold; baseline (speedup 1.0000x reference)
import jax
import jax.numpy as jnp
from jax.experimental import pallas as pl

_NOP = 50000
_HID = 128
_NB = 1024
_BX = 2000
_NBLK = _NOP // _BX
_ACCW = 8


def _body(ids_ref, x_ref, wt_ref, b_ref, g_ref, lb_ref, wmt_ref, bias_ref,
          out_ref):
    i = pl.program_id(0)

    @pl.when(i == 0)
    def _init():
        out_ref[...] = jnp.zeros_like(out_ref)

    h = jnp.dot(x_ref[...], wt_ref[...],
                preferred_element_type=jnp.float32) + b_ref[...]
    g = g_ref[...]
    lb = lb_ref[...]
    for _ in range(3):
        e = jnp.where(h > 0.0, h, jnp.expm1(h))
        m = jnp.mean(e, axis=1, keepdims=True)
        c = e - m
        v = jnp.mean(c * c, axis=1, keepdims=True)
        h = c * jax.lax.rsqrt(v + 1e-5) * g + lb

    p = jax.lax.dot_general(h, wmt_ref[...], (((1,), (1,)), ((), ())),
                            preferred_element_type=jnp.float32)
    cols = jax.lax.broadcasted_iota(jnp.int32, p.shape, 1)
    p = jnp.where(cols == 2, 1.0, p)

    ids = ids_ref[0, 0, :]
    onehot_t = (jax.lax.broadcasted_iota(jnp.int32, (_NB, _BX), 0)
                == ids[None, :]).astype(jnp.float32)
    out_ref[...] += jnp.dot(onehot_t, p,
                            preferred_element_type=jnp.float32)

    @pl.when(i == _NBLK - 1)
    def _fin():
        a = out_ref[...]
        cnt = jnp.clip(a[:, 2:3], 1.0, None)
        out_ref[...] = a / cnt + bias_ref[...]


def kernel(x_operator, W_operator, b_operator, x_table, W_table, b_table,
           x_column, W_column, b_column, x_predicate, W_predicate,
           b_predicate, x_operation, W_operation, b_operation, x_literal,
           W_literal, b_literal, x_numeral, W_numeral, b_numeral, ln_g, ln_b,
           W_mem, b_mem, W_time, b_time, batch_operator, ei_0, ei_1, ei_2,
           ei_3, ei_4, ei_5, ei_6, ei_7, ei_8, ei_9, ei_10, ei_11, ei_12,
           ei_13):
    f32 = jnp.float32
    wt = W_operator.T
    b_row = b_operator.reshape(1, _HID)
    g_row = ln_g.reshape(1, _HID)
    lb_row = ln_b.reshape(1, _HID)
    wmt = jnp.concatenate(
        [W_mem, W_time, jnp.zeros((_ACCW - 2, _HID), f32)], axis=0)
    bias_row = jnp.concatenate(
        [b_mem, b_time, jnp.zeros((_ACCW - 2,), f32)]).reshape(1, _ACCW)
    ids3 = batch_operator.reshape(_NBLK, 1, _BX)

    out = pl.pallas_call(
        _body,
        grid=(_NBLK,),
        in_specs=[
            pl.BlockSpec((1, 1, _BX), lambda i: (i, 0, 0)),
            pl.BlockSpec((_BX, 32), lambda i: (i, 0)),
            pl.BlockSpec((32, _HID), lambda i: (0, 0)),
            pl.BlockSpec((1, _HID), lambda i: (0, 0)),
            pl.BlockSpec((1, _HID), lambda i: (0, 0)),
            pl.BlockSpec((1, _HID), lambda i: (0, 0)),
            pl.BlockSpec((_ACCW, _HID), lambda i: (0, 0)),
            pl.BlockSpec((1, _ACCW), lambda i: (0, 0)),
        ],
        out_specs=pl.BlockSpec((_NB, _ACCW), lambda i: (0, 0)),
        out_shape=jax.ShapeDtypeStruct((_NB, _ACCW), f32),
    )(ids3, x_operator, wt, b_row, g_row, lb_row, wmt, bias_row)

    return (out[:, 0], out[:, 1])


# speedup vs baseline: 5.6566x; 5.6566x over previous
import jax
import jax.numpy as jnp
from jax.experimental import pallas as pl

_NOP = 50000
_HID = 128
_NB = 1024
_BX = 2000
_NBLK = _NOP // _BX
_ACCW = 8


def _body(ids_ref, x_ref, wt_ref, b_ref, g_ref, lb_ref, wmt_ref, bias_ref,
          out_ref):
    i = pl.program_id(0)

    @pl.when(i == 0)
    def _init():
        out_ref[...] = jnp.zeros_like(out_ref)

    h = jnp.dot(x_ref[...], wt_ref[...],
                preferred_element_type=jnp.float32) + b_ref[...]
    g = g_ref[...]
    lb = lb_ref[...]
    for _ in range(3):
        e = jnp.where(h > 0.0, h, jnp.exp(jnp.minimum(h, 0.0)) - 1.0)
        m = jnp.mean(e, axis=1, keepdims=True)
        c = e - m
        v = jnp.mean(c * c, axis=1, keepdims=True)
        h = c * jax.lax.rsqrt(v + 1e-5) * g + lb

    p = jax.lax.dot_general(h, wmt_ref[...], (((1,), (1,)), ((), ())),
                            preferred_element_type=jnp.float32)
    cols = jax.lax.broadcasted_iota(jnp.int32, p.shape, 1)
    p = jnp.where(cols == 2, 1.0, p)

    ids = ids_ref[0, 0, :]
    onehot_t = (jax.lax.broadcasted_iota(jnp.int32, (_NB, _BX), 0)
                == ids[None, :]).astype(jnp.float32)
    out_ref[...] += jnp.dot(onehot_t, p,
                            preferred_element_type=jnp.float32)

    @pl.when(i == _NBLK - 1)
    def _fin():
        a = out_ref[...]
        cnt = jnp.clip(a[:, 2:3], 1.0, None)
        out_ref[...] = a / cnt + bias_ref[...]


def kernel(x_operator, W_operator, b_operator, x_table, W_table, b_table,
           x_column, W_column, b_column, x_predicate, W_predicate,
           b_predicate, x_operation, W_operation, b_operation, x_literal,
           W_literal, b_literal, x_numeral, W_numeral, b_numeral, ln_g, ln_b,
           W_mem, b_mem, W_time, b_time, batch_operator, ei_0, ei_1, ei_2,
           ei_3, ei_4, ei_5, ei_6, ei_7, ei_8, ei_9, ei_10, ei_11, ei_12,
           ei_13):
    f32 = jnp.float32
    wt = W_operator.T
    b_row = b_operator.reshape(1, _HID)
    g_row = ln_g.reshape(1, _HID)
    lb_row = ln_b.reshape(1, _HID)
    wmt = jnp.concatenate(
        [W_mem, W_time, jnp.zeros((_ACCW - 2, _HID), f32)], axis=0)
    bias_row = jnp.concatenate(
        [b_mem, b_time, jnp.zeros((_ACCW - 2,), f32)]).reshape(1, _ACCW)
    ids3 = batch_operator.reshape(_NBLK, 1, _BX)

    out = pl.pallas_call(
        _body,
        grid=(_NBLK,),
        in_specs=[
            pl.BlockSpec((1, 1, _BX), lambda i: (i, 0, 0)),
            pl.BlockSpec((_BX, 32), lambda i: (i, 0)),
            pl.BlockSpec((32, _HID), lambda i: (0, 0)),
            pl.BlockSpec((1, _HID), lambda i: (0, 0)),
            pl.BlockSpec((1, _HID), lambda i: (0, 0)),
            pl.BlockSpec((1, _HID), lambda i: (0, 0)),
            pl.BlockSpec((_ACCW, _HID), lambda i: (0, 0)),
            pl.BlockSpec((1, _ACCW), lambda i: (0, 0)),
        ],
        out_specs=pl.BlockSpec((_NB, _ACCW), lambda i: (0, 0)),
        out_shape=jax.ShapeDtypeStruct((_NB, _ACCW), f32),
    )(ids3, x_operator, wt, b_row, g_row, lb_row, wmt, bias_row)

    return (out[:, 0], out[:, 1])
